# baseline (speedup 1.0000x reference)
import jax
import jax.numpy as jnp
from jax import lax
from jax.experimental import pallas as pl
from jax.experimental.pallas import tpu as pltpu
from jax.experimental.pallas import tpu_sc as plsc

NN = 10000
EE = 320000
DD = 128
BB = 64
HW = 64

NC = 2
NS = 16
DH = DD // NC
CH = 128
NCHUNK = 157
EPW = NCHUNK * CH
EPAD = NS * EPW - EE
NNP = 10240
RPS = NNP // NS
ZR = 32

NBUF = 4
LOOK = 2

_sc_mesh = plsc.VectorSubcoreMesh(
    core_axis_name="c", subcore_axis_name="s", num_cores=NC, num_subcores=NS)


def _make_segsum(with_deg):
  out_type = [jax.ShapeDtypeStruct((NC, NNP, DH), jnp.float32)]
  scratch = [
      pltpu.VMEM((NCHUNK, CH), jnp.int32),
      pltpu.VMEM((NCHUNK, CH), jnp.int32),
      [pltpu.VMEM((CH, DH), jnp.float32) for _ in range(NBUF)],
      pltpu.VMEM((ZR, DH), jnp.float32),
      pltpu.VMEM_SHARED((NNP, DH), jnp.float32),
      [pltpu.SemaphoreType.DMA for _ in range(NBUF)],
      [pltpu.SemaphoreType.DMA for _ in range(NBUF)],
  ]
  if with_deg:
    out_type.append(jax.ShapeDtypeStruct((NC, NNP, 16), jnp.float32))
    scratch += [
        pltpu.VMEM((CH, 16), jnp.float32),
        pltpu.VMEM((ZR, 16), jnp.float32),
        pltpu.VMEM_SHARED((NNP, 16), jnp.float32),
        [pltpu.SemaphoreType.DMA for _ in range(NBUF)],
    ]

  def body(table_hbm, src_hbm, dst_hbm, *refs):
    if with_deg:
      (out_hbm, deg_hbm, src_all, dst_all, rows, zero_v, acc_sh, sem_g,
       sem_s, ones_v, zdeg_v, deg_sh, sem_d) = refs
    else:
      out_hbm, src_all, dst_all, rows, zero_v, acc_sh, sem_g, sem_s = refs

    c = lax.axis_index("c")
    s = lax.axis_index("s")
    base_r = s * RPS

    def on_deg(i):
      return (jnp.asarray(i) + c) % 2 == 0

    z16 = jnp.zeros((16,), jnp.float32)

    def zrow(r, carry):
      for k in range(DH // 16):
        zero_v[r, k * 16:(k + 1) * 16] = z16
      return carry
    lax.fori_loop(0, ZR, zrow, 0)
    for j in range(RPS // ZR):
      pltpu.sync_copy(zero_v, acc_sh.at[pl.ds(base_r + j * ZR, ZR)])

    pltpu.sync_copy(src_hbm.at[s], src_all)
    pltpu.sync_copy(dst_hbm.at[s], dst_all)

    if with_deg:
      o16 = jnp.ones((16,), jnp.float32)

      def frow(r, carry):
        ones_v[r, :] = o16
        return carry
      lax.fori_loop(0, CH, frow, 0)

      def zdrow(r, carry):
        zdeg_v[r, :] = z16
        return carry
      lax.fori_loop(0, ZR, zdrow, 0)
      for j in range(RPS // ZR):
        pltpu.sync_copy(zdeg_v, deg_sh.at[pl.ds(base_r + j * ZR, ZR)])

    plsc.subcore_barrier()

    def issue_gather(i, k):
      pltpu.async_copy(table_hbm.at[c].at[src_all.at[i]], rows[k], sem_g[k])

    def wait_gather(i, k):
      pltpu.make_async_copy(table_hbm.at[c].at[src_all.at[i]], rows[k],
                            sem_g[k]).wait()

    def issue_scatter(i, k):
      pltpu.async_copy(rows[k], acc_sh.at[dst_all.at[i]], sem_s[k], add=True)
      if with_deg:
        @pl.when(on_deg(i))
        def _():
          pltpu.async_copy(ones_v, deg_sh.at[dst_all.at[i]], sem_d[k],
                           add=True)

    def wait_scatter(i, k):
      pltpu.make_async_copy(rows[k], acc_sh.at[dst_all.at[i]],
                            sem_s[k]).wait()
      if with_deg:
        @pl.when(on_deg(i))
        def _():
          pltpu.make_async_copy(ones_v, deg_sh.at[dst_all.at[i]],
                                sem_d[k]).wait()

    def step(i, k):
      wait_gather(i, k)
      issue_scatter(i, k)
      k2 = (k + LOOK) % NBUF
      wait_scatter(i - (NBUF - LOOK), k2)
      issue_gather(i + LOOK, k2)

    for i in range(LOOK):
      issue_gather(i, i % NBUF)
    for i in range(LOOK):
      k = i % NBUF
      wait_gather(i, k)
      issue_scatter(i, k)
      issue_gather(i + LOOK, (k + LOOK) % NBUF)

    n_main = (NCHUNK - LOOK - (NBUF - 1)) // NBUF
    tail0 = LOOK + n_main * NBUF

    def outer(j, carry):
      i0 = LOOK + j * NBUF
      for b in range(NBUF):
        step(i0 + b, (LOOK + b) % NBUF)
      return carry
    lax.fori_loop(0, n_main, outer, 0)

    for i in range(tail0, NCHUNK):
      k = i % NBUF
      wait_gather(i, k)
      issue_scatter(i, k)
      if i + LOOK < NCHUNK:
        k2 = (k + LOOK) % NBUF
        wait_scatter(i + LOOK - NBUF, k2)
        issue_gather(i + LOOK, k2)

    for i in range(NCHUNK - NBUF, NCHUNK):
      wait_scatter(i, i % NBUF)

    plsc.subcore_barrier()

    pltpu.sync_copy(acc_sh.at[pl.ds(base_r, RPS)],
                    out_hbm.at[c, pl.ds(base_r, RPS)])
    if with_deg:
      pltpu.sync_copy(deg_sh.at[pl.ds(base_r, RPS)],
                      deg_hbm.at[c, pl.ds(base_r, RPS)])

  out = tuple(out_type) if with_deg else out_type[0]
  return pl.kernel(body, out_type=out, mesh=_sc_mesh,
                   scratch_types=scratch,
                   compiler_params=pltpu.CompilerParams(
                       use_tc_tiling_on_sc=False),
                   name="segsum_deg" if with_deg else "segsum")


_segsum_deg = _make_segsum(True)
_segsum = _make_segsum(False)


ROWS_BLK = 1024
GRID = NNP // ROWS_BLK


def _full(shape):
  return pl.BlockSpec(shape, lambda i: (0,) * len(shape))


def _rows(w):
  return pl.BlockSpec((ROWS_BLK, w), lambda i: (i, 0))


def _stk(w):
  return pl.BlockSpec((NC, ROWS_BLK, w), lambda i: (0, i, 0))


def _dotT(a, w):
  return lax.dot_general(a, w, (((1,), (1,)), ((), ())),
                         preferred_element_type=jnp.float32)


def _tc_pre_body(x_ref, wl_ref, wr_ref, bl_ref, tbl_ref, xr_ref):
  xb = x_ref[...]
  xl = _dotT(xb, wl_ref[...])
  tbl_ref[0] = xl[:, :DH]
  tbl_ref[1] = xl[:, DH:]
  xr_ref[...] = _dotT(xb, wr_ref[...]) + bl_ref[...]


_tc_pre = pl.pallas_call(
    _tc_pre_body,
    grid=(GRID,),
    in_specs=[_rows(DD), _full((DD, DD)), _full((DD, DD)), _full((1, DD))],
    out_specs=[_stk(DH), _rows(DD)],
    out_shape=[jax.ShapeDtypeStruct((NC, NNP, DH), jnp.float32),
               jax.ShapeDtypeStruct((NNP, DD), jnp.float32)],
)


def _tc_mid_body(p_ref, d_ref, xr1_ref, wl_ref, wr_ref,
                 bl_ref, tbl2_ref, xr2_ref, dinv_ref):
  p = jnp.concatenate([p_ref[0], p_ref[1]], axis=1)
  deg = d_ref[0][:, :1] + d_ref[1][:, :1]
  dinv = 1.0 / jnp.maximum(deg, 1.0)
  h1 = jnp.maximum(p * dinv + xr1_ref[...], 0.0)
  xl2 = _dotT(h1, wl_ref[...])
  tbl2_ref[0] = xl2[:, :DH]
  tbl2_ref[1] = xl2[:, DH:]
  xr2_ref[...] = _dotT(h1, wr_ref[...]) + bl_ref[...]
  dinv_ref[...] = jnp.broadcast_to(dinv, (ROWS_BLK, 8))


_tc_mid = pl.pallas_call(
    _tc_mid_body,
    grid=(GRID,),
    in_specs=[_stk(DH), _stk(16), _rows(DD),
              _full((DD, DD)), _full((DD, DD)), _full((1, DD))],
    out_specs=[_stk(DH), _rows(DD), _rows(8)],
    out_shape=[jax.ShapeDtypeStruct((NC, NNP, DH), jnp.float32),
               jax.ShapeDtypeStruct((NNP, DD), jnp.float32),
               jax.ShapeDtypeStruct((NNP, 8), jnp.float32)],
)


def _tc_head_body(q_ref, xr2_ref, dinv_ref, bv_ref, ltni_ref,
                  wc1_ref, bc1_ref, wc2_ref, bc2_ref, out_ref):
  q = jnp.concatenate([q_ref[0], q_ref[1]], axis=1)
  h2 = jnp.maximum(q * dinv_ref[...][:, :1]
                   + xr2_ref[...], 0.0)
  bv = bv_ref[...]
  iota_b = lax.broadcasted_iota(jnp.int32, (BB, 1), 0)
  cmp = (bv < iota_b).astype(jnp.int32)
  offs = jnp.sum(cmp, axis=1, keepdims=True) + ltni_ref[...]
  iota_n = lax.broadcasted_iota(jnp.int32, (1, NNP), 1)
  onehot = (offs == iota_n).astype(jnp.float32)
  tgt = lax.dot_general(onehot, h2, (((1,), (0,)), ((), ())),
                        preferred_element_type=jnp.float32)
  z = jnp.maximum(_dotT(tgt, wc1_ref[...]) + bc1_ref[...], 0.0)
  out_ref[...] = jnp.sum(z * wc2_ref[...], axis=1, keepdims=True) + bc2_ref[...]


_tc_head = pl.pallas_call(
    _tc_head_body,
    grid=(1,),
    in_specs=[_full((NC, NNP, DH)), _full((NNP, DD)),
              _full((NNP, 8)), _full((1, NNP)), _full((BB, 1)),
              _full((HW, DD)), _full((1, HW)),
              _full((1, HW)), _full((BB, 1))],
    out_specs=_full((BB, 1)),
    out_shape=jax.ShapeDtypeStruct((BB, 1), jnp.float32),
)


@jax.jit
def kernel(x, edge_index, local_target_node_idx, batch_vector,
           Wl1, bl1, Wr1, Wl2, bl2, Wr2, Wc1, bc1, Wc2, bc2):
  ei = edge_index.astype(jnp.int32)
  src = jnp.concatenate(
      [ei[0], jnp.zeros((EPAD,), jnp.int32)]).reshape(NS, NCHUNK, CH)
  dst = jnp.concatenate(
      [ei[1], jnp.full((EPAD,), NN, jnp.int32)]).reshape(NS, NCHUNK, CH)
  xp = jnp.pad(x, ((0, NNP - NN), (0, 0)))
  bvp = jnp.pad(batch_vector.astype(jnp.int32), (0, NNP - NN),
                constant_values=BB)

  tbl1, xr1 = _tc_pre(xp, Wl1, Wr1, bl1.reshape(1, DD))
  p1, deg = _segsum_deg(tbl1, src, dst)
  tbl2, xr2, dinv = _tc_mid(p1, deg, xr1,
                            Wl2, Wr2, bl2.reshape(1, DD))
  p2 = _segsum(tbl2, src, dst)
  out = _tc_head(p2, xr2, dinv,
                 bvp.reshape(1, NNP),
                 local_target_node_idx.reshape(BB, 1).astype(jnp.int32),
                 Wc1, bc1.reshape(1, HW), Wc2.reshape(1, HW),
                 jnp.broadcast_to(bc2.reshape(1, 1), (BB, 1)))
  return out


# speedup vs baseline: 1.3842x; 1.3842x over previous
import jax
import jax.numpy as jnp
from jax import lax
from jax.experimental import pallas as pl
from jax.experimental.pallas import tpu as pltpu
from jax.experimental.pallas import tpu_sc as plsc

NN = 10000
EE = 320000
DD = 128
BB = 64
HW = 64

NC = 2
NS = 16
DH = DD // NC
EPW = EE // NS
CH = 80
NCHUNK = EPW // CH
RPS = NN // NS
ZR = 25

NBUF = 6
LOOK = 3

_sc_mesh = plsc.VectorSubcoreMesh(
    core_axis_name="c", subcore_axis_name="s", num_cores=NC, num_subcores=NS)


def _make_segsum(with_deg):
  out_type = [jax.ShapeDtypeStruct((NN, DD), jnp.float32)]
  scratch = [
      pltpu.VMEM((NCHUNK, CH), jnp.int32),
      pltpu.VMEM((NCHUNK, CH), jnp.int32),
      [pltpu.VMEM((CH, DH), jnp.float32) for _ in range(NBUF)],
      pltpu.VMEM((ZR, DH), jnp.float32),
      pltpu.VMEM_SHARED((NN, DH), jnp.float32),
      [pltpu.SemaphoreType.DMA for _ in range(NBUF)],
      [pltpu.SemaphoreType.DMA for _ in range(NBUF)],
  ]
  if with_deg:
    out_type.append(jax.ShapeDtypeStruct((NN, 16), jnp.float32))
    scratch += [
        pltpu.VMEM((CH, 16), jnp.float32),
        pltpu.VMEM((ZR, 16), jnp.float32),
        pltpu.VMEM_SHARED((NN, 16), jnp.float32),
        [pltpu.SemaphoreType.DMA for _ in range(NBUF)],
    ]

  def body(table_hbm, src_hbm, dst_hbm, *refs):
    if with_deg:
      (out_hbm, deg_hbm, src_all, dst_all, rows, zero_v, acc_sh, sem_g,
       sem_s, ones_v, zdeg_v, deg_sh, sem_d) = refs
    else:
      out_hbm, src_all, dst_all, rows, zero_v, acc_sh, sem_g, sem_s = refs

    c = lax.axis_index("c")
    s = lax.axis_index("s")
    base_r = s * RPS
    on_deg_core = c == 0

    z16 = jnp.zeros((16,), jnp.float32)

    def zrow(r, carry):
      for k in range(DH // 16):
        zero_v[r, k * 16:(k + 1) * 16] = z16
      return carry
    lax.fori_loop(0, ZR, zrow, 0)
    for j in range(RPS // ZR):
      pltpu.sync_copy(zero_v, acc_sh.at[pl.ds(base_r + j * ZR, ZR)])

    pltpu.sync_copy(src_hbm.at[s], src_all)
    pltpu.sync_copy(dst_hbm.at[s], dst_all)

    @pl.when(c == 1)
    def _():
      def adj(r, carry):
        for kk in range(CH // 16):
          sl = slice(kk * 16, (kk + 1) * 16)
          src_all[r, sl] = src_all[r, sl] + 1
        return carry
      lax.fori_loop(0, NCHUNK, adj, 0)

    if with_deg:
      @pl.when(on_deg_core)
      def _():
        o16 = jnp.ones((16,), jnp.float32)

        def frow(r, carry):
          ones_v[r, :] = o16
          return carry
        lax.fori_loop(0, CH, frow, 0)

        def zdrow(r, carry):
          zdeg_v[r, :] = z16
          return carry
        lax.fori_loop(0, ZR, zdrow, 0)
        for j in range(RPS // ZR):
          pltpu.sync_copy(zdeg_v, deg_sh.at[pl.ds(base_r + j * ZR, ZR)])

    plsc.subcore_barrier()

    def issue_gather(i, k):
      pltpu.async_copy(table_hbm.at[src_all.at[i]], rows[k], sem_g[k])

    def wait_gather(i, k):
      pltpu.make_async_copy(table_hbm.at[src_all.at[i]], rows[k],
                            sem_g[k]).wait()

    def issue_scatter(i, k):
      pltpu.async_copy(rows[k], acc_sh.at[dst_all.at[i]], sem_s[k], add=True)
      if with_deg:
        @pl.when(on_deg_core)
        def _():
          pltpu.async_copy(ones_v, deg_sh.at[dst_all.at[i]], sem_d[k],
                           add=True)

    def wait_scatter(i, k):
      pltpu.make_async_copy(rows[k], acc_sh.at[dst_all.at[i]],
                            sem_s[k]).wait()
      if with_deg:
        @pl.when(on_deg_core)
        def _():
          pltpu.make_async_copy(ones_v, deg_sh.at[dst_all.at[i]],
                                sem_d[k]).wait()

    def step(i, k):
      wait_gather(i, k)
      issue_scatter(i, k)
      k2 = (k + LOOK) % NBUF
      wait_scatter(i - LOOK, k2)
      issue_gather(i + LOOK, k2)

    for i in range(LOOK):
      issue_gather(i, i % NBUF)
    for i in range(LOOK):
      k = i % NBUF
      wait_gather(i, k)
      issue_scatter(i, k)
      issue_gather(i + LOOK, (k + LOOK) % NBUF)

    n_main = (NCHUNK - LOOK - (NBUF - 1)) // NBUF
    tail0 = LOOK + n_main * NBUF

    def outer(j, carry):
      i0 = LOOK + j * NBUF
      for b in range(NBUF):
        step(i0 + b, (LOOK + b) % NBUF)
      return carry
    lax.fori_loop(0, n_main, outer, 0)

    for i in range(tail0, NCHUNK):
      k = i % NBUF
      wait_gather(i, k)
      issue_scatter(i, k)
      if i + LOOK < NCHUNK:
        k2 = (k + LOOK) % NBUF
        wait_scatter(i + LOOK - NBUF, k2)
        issue_gather(i + LOOK, k2)

    for i in range(NCHUNK - NBUF, NCHUNK):
      wait_scatter(i, i % NBUF)

    plsc.subcore_barrier()

    pltpu.sync_copy(acc_sh.at[pl.ds(base_r, RPS)],
                    out_hbm.at[pl.ds(base_r, RPS), pl.ds(c * DH, DH)])
    if with_deg:
      @pl.when(on_deg_core)
      def _():
        pltpu.sync_copy(deg_sh.at[pl.ds(base_r, RPS)],
                        deg_hbm.at[pl.ds(base_r, RPS)])

  out = tuple(out_type) if with_deg else out_type[0]
  return pl.kernel(body, out_type=out, mesh=_sc_mesh,
                   scratch_types=scratch,
                   compiler_params=pltpu.CompilerParams(
                       use_tc_tiling_on_sc=False),
                   name="segsum_deg" if with_deg else "segsum")


_segsum_deg = _make_segsum(True)
_segsum = _make_segsum(False)


ROWS_BLK = 1000
GRID = NN // ROWS_BLK


def _full(shape):
  return pl.BlockSpec(shape, lambda i: (0,) * len(shape))


def _rows(w):
  return pl.BlockSpec((ROWS_BLK, w), lambda i: (i, 0))


def _dotT(a, w):
  return lax.dot_general(a, w, (((1,), (1,)), ((), ())),
                         preferred_element_type=jnp.float32)


def _tbl_spec():
  return pl.BlockSpec((NC, ROWS_BLK, DH), lambda i: (0, i, 0))


def _tc_pre_body(x_ref, wl_ref, wr_ref, bl_ref, tbl_ref, xr_ref):
  xb = x_ref[...]
  tbl_ref[...] = _dotT(xb, wl_ref[...])
  xr_ref[...] = _dotT(xb, wr_ref[...]) + bl_ref[...]


_tc_pre = pl.pallas_call(
    _tc_pre_body,
    grid=(GRID,),
    in_specs=[_rows(DD), _full((DD, DD)), _full((DD, DD)), _full((1, DD))],
    out_specs=[_rows(DD), _rows(DD)],
    out_shape=[jax.ShapeDtypeStruct((NN, DD), jnp.float32),
               jax.ShapeDtypeStruct((NN, DD), jnp.float32)],
)


def _tc_mid_body(p_ref, d_ref, xr1_ref, wl_ref, wr_ref,
                 bl_ref, tbl2_ref, xr2_ref, dinv_ref):
  deg = d_ref[...][:, :1]
  dinv = 1.0 / jnp.maximum(deg, 1.0)
  h1 = jnp.maximum(p_ref[...] * dinv + xr1_ref[...], 0.0)
  tbl2_ref[...] = _dotT(h1, wl_ref[...])
  xr2_ref[...] = _dotT(h1, wr_ref[...]) + bl_ref[...]
  dinv_ref[...] = jnp.broadcast_to(dinv, (ROWS_BLK, 8))


_tc_mid = pl.pallas_call(
    _tc_mid_body,
    grid=(GRID,),
    in_specs=[_rows(DD), _rows(16), _rows(DD),
              _full((DD, DD)), _full((DD, DD)), _full((1, DD))],
    out_specs=[_rows(DD), _rows(DD), _rows(8)],
    out_shape=[jax.ShapeDtypeStruct((NN, DD), jnp.float32),
               jax.ShapeDtypeStruct((NN, DD), jnp.float32),
               jax.ShapeDtypeStruct((NN, 8), jnp.float32)],
)


def _tc_head_body(q_ref, xr2_ref, dinv_ref, bv_ref, ltni_ref,
                  wc1_ref, bc1_ref, wc2_ref, bc2_ref, out_ref):
  h2 = jnp.maximum(q_ref[...] * dinv_ref[...][:, :1]
                   + xr2_ref[...], 0.0)
  bv = bv_ref[...]
  iota_b = lax.broadcasted_iota(jnp.int32, (BB, 1), 0)
  cmp = (bv < iota_b).astype(jnp.int32)
  offs = jnp.sum(cmp, axis=1, keepdims=True) + ltni_ref[...]
  iota_n = lax.broadcasted_iota(jnp.int32, (1, NN), 1)
  onehot = (offs == iota_n).astype(jnp.float32)
  tgt = lax.dot_general(onehot, h2, (((1,), (0,)), ((), ())),
                        preferred_element_type=jnp.float32)
  z = jnp.maximum(_dotT(tgt, wc1_ref[...]) + bc1_ref[...], 0.0)
  out_ref[...] = jnp.sum(z * wc2_ref[...], axis=1, keepdims=True) + bc2_ref[...]


_tc_head = pl.pallas_call(
    _tc_head_body,
    grid=(1,),
    in_specs=[_full((NN, DD)), _full((NN, DD)),
              _full((NN, 8)), _full((1, NN)), _full((BB, 1)),
              _full((HW, DD)), _full((1, HW)),
              _full((1, HW)), _full((BB, 1))],
    out_specs=_full((BB, 1)),
    out_shape=jax.ShapeDtypeStruct((BB, 1), jnp.float32),
)


@jax.jit
def kernel(x, edge_index, local_target_node_idx, batch_vector,
           Wl1, bl1, Wr1, Wl2, bl2, Wr2, Wc1, bc1, Wc2, bc2):
  src = (edge_index[0].astype(jnp.int32) * 2).reshape(NS, NCHUNK, CH)
  dst = edge_index[1].astype(jnp.int32).reshape(NS, NCHUNK, CH)

  tbl1, xr1 = _tc_pre(x, Wl1, Wr1, bl1.reshape(1, DD))
  p1, deg = _segsum_deg(tbl1.reshape(2 * NN, DH), src, dst)
  tbl2, xr2, dinv = _tc_mid(p1, deg, xr1,
                            Wl2, Wr2, bl2.reshape(1, DD))
  p2 = _segsum(tbl2.reshape(2 * NN, DH), src, dst)
  out = _tc_head(p2, xr2, dinv,
                 batch_vector.reshape(1, NN).astype(jnp.int32),
                 local_target_node_idx.reshape(BB, 1).astype(jnp.int32),
                 Wc1, bc1.reshape(1, HW), Wc2.reshape(1, HW),
                 jnp.broadcast_to(bc2.reshape(1, 1), (BB, 1)))
  return out
